# baseline (speedup 1.0000x reference)
import functools

import jax
import jax.numpy as jnp
from jax import lax
from jax.experimental import pallas as pl
from jax.experimental.pallas import tpu as pltpu
from jax.experimental.pallas import tpu_sc as plsc


@functools.cache
def _make_transpose(D, N, C):
    grid = (N + C - 1) // C

    def body(x_ref, o_ref):
        o_ref[...] = x_ref[...].T

    return pl.pallas_call(
        body,
        grid=(grid,),
        in_specs=[pl.BlockSpec((D, C), lambda i: (0, i))],
        out_specs=pl.BlockSpec((C, D), lambda i: (i, 0)),
        out_shape=jax.ShapeDtypeStruct((N, D), jnp.float32),
        compiler_params=pltpu.CompilerParams(
            vmem_limit_bytes=100 * 1024 * 1024),
    )


@functools.cache
def _make_untranspose(D, B, C):
    grid = B // C

    def body(h_ref, t_ref, r_ref, ho_ref, to_ref, ro_ref):
        ho_ref[...] = h_ref[...].T
        to_ref[...] = t_ref[...].T
        ro_ref[...] = r_ref[...].T

    inspec = pl.BlockSpec((C, D), lambda i: (i, 0))
    outspec = pl.BlockSpec((D, C), lambda i: (0, i))
    sh = jax.ShapeDtypeStruct((D, B), jnp.float32)
    return pl.pallas_call(
        body,
        grid=(grid,),
        in_specs=[inspec, inspec, inspec],
        out_specs=[outspec, outspec, outspec],
        out_shape=(sh, sh, sh),
    )


@functools.cache
def _make_gather_ent(NE, D, B):
    info = plsc.get_sparse_core_info()
    NC, NS = info.num_cores, info.num_subcores
    NW = NC * NS
    assert B % (8 * NW) == 0
    bpw = B // NW
    mesh = plsc.VectorSubcoreMesh(core_axis_name="c", subcore_axis_name="s")
    f32 = jnp.float32
    out_row = jax.ShapeDtypeStruct((B, D), f32)

    @functools.partial(
        pl.kernel,
        mesh=mesh,
        out_type=(out_row, out_row),
        scratch_types=[
            pltpu.VMEM((bpw,), jnp.int32),
            pltpu.VMEM((bpw,), jnp.int32),
            pltpu.SemaphoreType.DMA,
            pltpu.SemaphoreType.DMA,
        ],
    )
    def k(ent_hbm, h_hbm, t_hbm, ho_hbm, to_hbm,
          hi_v, ti_v, sem_h, sem_t):
        wid = lax.axis_index("s") * NC + lax.axis_index("c")
        base = wid * bpw
        pltpu.sync_copy(h_hbm.at[pl.ds(base, bpw)], hi_v)
        pltpu.sync_copy(t_hbm.at[pl.ds(base, bpw)], ti_v)

        def fire(idx_v, o_hbm, sem):
            def body(c, carry):
                ivec = idx_v[pl.ds(c * 16, 16)]
                for j in range(16):
                    pltpu.async_copy(
                        ent_hbm.at[pl.ds(ivec[j], 1), :],
                        o_hbm.at[pl.ds(base + c * 16 + j, 1), :],
                        sem,
                    )
                return carry
            lax.fori_loop(0, bpw // 16, body, 0)

        fire(hi_v, ho_hbm, sem_h)
        fire(ti_v, to_hbm, sem_t)
        pltpu.make_async_copy(
            ent_hbm.at[pl.ds(0, bpw), :], ho_hbm.at[pl.ds(base, bpw)],
            sem_h).wait()
        pltpu.make_async_copy(
            ent_hbm.at[pl.ds(0, bpw), :], to_hbm.at[pl.ds(base, bpw)],
            sem_t).wait()

    return k


@functools.cache
def _make_gather_rel(NR, D, B):
    info = plsc.get_sparse_core_info()
    NC, NS = info.num_cores, info.num_subcores
    NW = NC * NS
    bpw = B // NW
    mesh = plsc.VectorSubcoreMesh(core_axis_name="c", subcore_axis_name="s")
    f32 = jnp.float32

    @functools.partial(
        pl.kernel,
        mesh=mesh,
        out_type=jax.ShapeDtypeStruct((B, D), f32),
        scratch_types=[
            pltpu.VMEM((bpw,), jnp.int32),
            pltpu.VMEM((bpw, D), f32),
            pltpu.SemaphoreType.DMA,
            pltpu.SemaphoreType.DMA,
        ],
    )
    def k(rel_hbm, r_hbm, ro_hbm, idx_v, rows_v, sem, sem_out):
        wid = lax.axis_index("s") * NC + lax.axis_index("c")
        base = wid * bpw
        pltpu.sync_copy(r_hbm.at[pl.ds(base, bpw)], idx_v)

        def body(c, carry):
            ivec = idx_v[pl.ds(c * 16, 16)]
            for j in range(16):
                pltpu.async_copy(
                    rel_hbm.at[pl.ds(ivec[j], 1), :],
                    rows_v.at[pl.ds(c * 16 + j, 1), :],
                    sem,
                )
            return carry
        lax.fori_loop(0, bpw // 16, body, 0)
        pltpu.make_async_copy(rel_hbm.at[pl.ds(0, bpw), :], rows_v, sem).wait()
        pltpu.async_copy(rows_v, ro_hbm.at[pl.ds(base, bpw)], sem_out).wait()

    return k


def kernel(entity_table, relation_table, h, t, r):
    B = h.shape[0]
    NE, D = entity_table.shape
    NR = relation_table.shape[0]
    ent_rm = _make_transpose(D, NE, 40960)(jnp.transpose(entity_table))
    ro = _make_gather_rel(NR, D, B)(relation_table, r.astype(jnp.int32))
    ho, to = _make_gather_ent(NE, D, B)(
        ent_rm, h.astype(jnp.int32), t.astype(jnp.int32))
    hoT, toT, roT = _make_untranspose(D, B, 8192)(ho, to, ro)
    return (jnp.transpose(hoT), jnp.transpose(toT), jnp.transpose(roT))


# speedup vs baseline: 2.6613x; 2.6613x over previous
import functools

import jax
import jax.numpy as jnp
from jax import lax
from jax.experimental import pallas as pl
from jax.experimental.pallas import tpu as pltpu
from jax.experimental.pallas import tpu_sc as plsc


@functools.cache
def _make_transpose(D, N, C):
    grid = (N + C - 1) // C

    def body(x_ref, o_ref):
        o_ref[...] = x_ref[...].T

    return pl.pallas_call(
        body,
        grid=(grid,),
        in_specs=[pl.BlockSpec((D, C), lambda i: (0, i))],
        out_specs=pl.BlockSpec((C, D), lambda i: (i, 0)),
        out_shape=jax.ShapeDtypeStruct((N, D), jnp.float32),
        compiler_params=pltpu.CompilerParams(
            vmem_limit_bytes=100 * 1024 * 1024),
    )


@functools.cache
def _make_untranspose(D, B, C):
    grid = B // C

    def body(h_ref, t_ref, r_ref, ho_ref, to_ref, ro_ref):
        ho_ref[...] = h_ref[...].T
        to_ref[...] = t_ref[...].T
        ro_ref[...] = r_ref[...].T

    inspec = pl.BlockSpec((C, D), lambda i: (i, 0))
    outspec = pl.BlockSpec((D, C), lambda i: (0, i))
    sh = jax.ShapeDtypeStruct((D, B), jnp.float32)
    return pl.pallas_call(
        body,
        grid=(grid,),
        in_specs=[inspec, inspec, inspec],
        out_specs=[outspec, outspec, outspec],
        out_shape=(sh, sh, sh),
    )


@functools.cache
def _make_gather_ent(NE, D, B):
    info = plsc.get_sparse_core_info()
    NC, NS = info.num_cores, info.num_subcores
    NW = NC * NS
    assert B % (8 * NW) == 0
    bpw = B // NW
    mesh = plsc.VectorSubcoreMesh(core_axis_name="c", subcore_axis_name="s")
    f32 = jnp.float32
    out_row = jax.ShapeDtypeStruct((B, D), f32)

    @functools.partial(
        pl.kernel,
        mesh=mesh,
        out_type=(out_row, out_row),
        scratch_types=[
            pltpu.VMEM((bpw,), jnp.int32),
            pltpu.VMEM((bpw, D), f32),
            pltpu.SemaphoreType.DMA,
            pltpu.SemaphoreType.DMA,
        ],
    )
    def k(ent_hbm, h_hbm, t_hbm, ho_hbm, to_hbm,
          idx_v, rows_v, sem, sem_out):
        wid = lax.axis_index("s") * NC + lax.axis_index("c")
        base = wid * bpw

        def gather_one(i_hbm, o_hbm):
            pltpu.sync_copy(i_hbm.at[pl.ds(base, bpw)], idx_v)

            def body(c, carry):
                ivec = idx_v[pl.ds(c * 16, 16)]
                for j in range(16):
                    pltpu.async_copy(
                        ent_hbm.at[pl.ds(ivec[j], 1), :],
                        rows_v.at[pl.ds(c * 16 + j, 1), :],
                        sem,
                    )
                return carry

            lax.fori_loop(0, bpw // 16, body, 0)
            pltpu.make_async_copy(
                ent_hbm.at[pl.ds(0, bpw), :], rows_v, sem).wait()
            return pltpu.async_copy(
                rows_v, o_hbm.at[pl.ds(base, bpw)], sem_out)

        c1 = gather_one(h_hbm, ho_hbm)
        c1.wait()
        c2 = gather_one(t_hbm, to_hbm)
        c2.wait()

    return k


@functools.cache
def _make_gather_rel(NR, D, B):
    info = plsc.get_sparse_core_info()
    NC, NS = info.num_cores, info.num_subcores
    NW = NC * NS
    bpw = B // NW
    mesh = plsc.VectorSubcoreMesh(core_axis_name="c", subcore_axis_name="s")
    f32 = jnp.float32

    @functools.partial(
        pl.kernel,
        mesh=mesh,
        out_type=jax.ShapeDtypeStruct((B, D), f32),
        scratch_types=[
            pltpu.VMEM((bpw,), jnp.int32),
            pltpu.VMEM((bpw, D), f32),
            pltpu.SemaphoreType.DMA,
            pltpu.SemaphoreType.DMA,
        ],
    )
    def k(rel_hbm, r_hbm, ro_hbm, idx_v, rows_v, sem, sem_out):
        wid = lax.axis_index("s") * NC + lax.axis_index("c")
        base = wid * bpw
        pltpu.sync_copy(r_hbm.at[pl.ds(base, bpw)], idx_v)

        def body(c, carry):
            ivec = idx_v[pl.ds(c * 16, 16)]
            for j in range(16):
                pltpu.async_copy(
                    rel_hbm.at[pl.ds(ivec[j], 1), :],
                    rows_v.at[pl.ds(c * 16 + j, 1), :],
                    sem,
                )
            return carry
        lax.fori_loop(0, bpw // 16, body, 0)
        pltpu.make_async_copy(rel_hbm.at[pl.ds(0, bpw), :], rows_v, sem).wait()
        pltpu.async_copy(rows_v, ro_hbm.at[pl.ds(base, bpw)], sem_out).wait()

    return k


def kernel(entity_table, relation_table, h, t, r):
    B = h.shape[0]
    NE, D = entity_table.shape
    NR = relation_table.shape[0]
    ent_rm = _make_transpose(D, NE, 40960)(jnp.transpose(entity_table))
    ro = _make_gather_rel(NR, D, B)(relation_table, r.astype(jnp.int32))
    ho, to = _make_gather_ent(NE, D, B)(
        ent_rm, h.astype(jnp.int32), t.astype(jnp.int32))
    hoT, toT, roT = _make_untranspose(D, B, 8192)(ho, to, ro)
    return (jnp.transpose(hoT), jnp.transpose(toT), jnp.transpose(roT))


# speedup vs baseline: 2.6947x; 1.0125x over previous
import functools

import jax
import jax.numpy as jnp
from jax import lax
from jax.experimental import pallas as pl
from jax.experimental.pallas import tpu as pltpu
from jax.experimental.pallas import tpu_sc as plsc


@functools.cache
def _make_transpose(D, N, C):
    grid = (N + C - 1) // C

    def body(x_ref, o_ref):
        o_ref[...] = x_ref[...].T

    return pl.pallas_call(
        body,
        grid=(grid,),
        in_specs=[pl.BlockSpec((D, C), lambda i: (0, i))],
        out_specs=pl.BlockSpec((C, D), lambda i: (i, 0)),
        out_shape=jax.ShapeDtypeStruct((N, D), jnp.float32),
        compiler_params=pltpu.CompilerParams(
            vmem_limit_bytes=100 * 1024 * 1024),
    )


@functools.cache
def _make_untranspose(D, B, C):
    grid = B // C

    def body(h_ref, t_ref, r_ref, ho_ref, to_ref, ro_ref):
        ho_ref[...] = h_ref[...].T
        to_ref[...] = t_ref[...].T
        ro_ref[...] = r_ref[...].T

    inspec = pl.BlockSpec((C, D), lambda i: (i, 0))
    outspec = pl.BlockSpec((D, C), lambda i: (0, i))
    sh = jax.ShapeDtypeStruct((D, B), jnp.float32)
    return pl.pallas_call(
        body,
        grid=(grid,),
        in_specs=[inspec, inspec, inspec],
        out_specs=[outspec, outspec, outspec],
        out_shape=(sh, sh, sh),
    )


@functools.cache
def _make_gather(NE, NR, D, B):
    info = plsc.get_sparse_core_info()
    NC, NS = info.num_cores, info.num_subcores
    NW = NC * NS
    assert B % (8 * NW) == 0
    bpw = B // NW
    mesh = plsc.VectorSubcoreMesh(core_axis_name="c", subcore_axis_name="s")
    f32 = jnp.float32
    out_row = jax.ShapeDtypeStruct((B, D), f32)

    @functools.partial(
        pl.kernel,
        mesh=mesh,
        out_type=(out_row, out_row, out_row),
        scratch_types=[
            pltpu.VMEM((bpw,), jnp.int32),
            pltpu.VMEM((bpw, D), f32),
            pltpu.SemaphoreType.DMA,
            pltpu.SemaphoreType.DMA,
        ],
    )
    def k(ent_hbm, rel_hbm, h_hbm, t_hbm, r_hbm,
          ho_hbm, to_hbm, ro_hbm,
          idx_v, rows_v, sem, sem_out):
        wid = lax.axis_index("s") * NC + lax.axis_index("c")
        base = wid * bpw

        def gather_one(tab_hbm, i_hbm, o_hbm):
            pltpu.sync_copy(i_hbm.at[pl.ds(base, bpw)], idx_v)

            def body(c, carry):
                ivec = idx_v[pl.ds(c * 16, 16)]
                for j in range(16):
                    pltpu.async_copy(
                        tab_hbm.at[pl.ds(ivec[j], 1), :],
                        rows_v.at[pl.ds(c * 16 + j, 1), :],
                        sem,
                    )
                return carry

            lax.fori_loop(0, bpw // 16, body, 0)
            pltpu.make_async_copy(
                tab_hbm.at[pl.ds(0, bpw), :], rows_v, sem).wait()
            return pltpu.async_copy(
                rows_v, o_hbm.at[pl.ds(base, bpw)], sem_out)

        c1 = gather_one(ent_hbm, h_hbm, ho_hbm)
        c1.wait()
        c2 = gather_one(ent_hbm, t_hbm, to_hbm)
        c2.wait()
        c3 = gather_one(rel_hbm, r_hbm, ro_hbm)
        c3.wait()

    return k


def kernel(entity_table, relation_table, h, t, r):
    B = h.shape[0]
    NE, D = entity_table.shape
    NR = relation_table.shape[0]
    ent_rm = _make_transpose(D, NE, 40960)(jnp.transpose(entity_table))
    ho, to, ro = _make_gather(NE, NR, D, B)(
        ent_rm, relation_table,
        h.astype(jnp.int32), t.astype(jnp.int32), r.astype(jnp.int32))
    hoT, toT, roT = _make_untranspose(D, B, 8192)(ho, to, ro)
    return (jnp.transpose(hoT), jnp.transpose(toT), jnp.transpose(roT))
